# baseline (speedup 1.0000x reference)
import functools

import jax
import jax.numpy as jnp
from jax import lax
from jax.experimental import pallas as pl
from jax.experimental.pallas import tpu as pltpu
from jax.experimental.pallas import tpu_sc as plsc

_IDS = [0, 195, 13, 182, 90, 110, 175, 6, 84, 45, 51, 129, 135, 69, 186, 3,
        9, 42, 48, 87, 93, 126, 132, 152, 192, 25, 81, 155, 159, 41, 53, 157,
        163, 184, 15, 18, 21, 30, 33, 36, 38, 57, 60, 63, 66, 72, 75, 78, 97,
        99, 102, 105, 108, 114, 117, 120, 123, 125, 142, 144, 147, 150, 165,
        167, 180, 188, 190, 1, 2, 4, 5, 7, 8, 10, 11, 12, 14, 16, 17, 19, 20,
        22, 23, 24, 26, 27, 28, 29, 31, 32, 34, 35, 37, 39, 40, 43, 44, 46,
        47, 49, 50, 52, 54, 55, 56, 58, 59, 61, 62, 64, 65, 67, 68, 70, 71,
        73, 74, 76, 77, 79, 80, 82, 83, 85, 86, 88, 89, 91, 92, 94, 95, 96,
        98, 100, 101, 103, 104, 106, 107, 109, 111, 112, 113, 115, 116, 118,
        119, 121, 122, 124, 127, 128, 130, 131, 133, 134, 136, 137, 138, 139,
        140, 141, 143, 145, 146, 148, 149, 151, 153, 154, 156, 158, 160, 161,
        162, 164, 166, 168, 169, 170, 171, 172, 173, 174, 176, 177, 178, 179,
        181, 183, 185, 187, 189, 191, 193, 194]


def _contiguous_runs(ids):
    runs = []
    j = 0
    while j < len(ids):
        a = ids[j]
        n = 1
        while j + n < len(ids) and ids[j + n] == a + n:
            n += 1
        runs.append((j, a, n))
        j += n
    return runs


def _assign(runs, num_workers):
    bins = [[] for _ in range(num_workers)]
    loads = [0] * num_workers
    for run in sorted(runs, key=lambda r: -r[2]):
        w = loads.index(min(loads))
        bins[w].append(run)
        loads[w] += run[2]
    return bins


_NUM_WORKERS = 32
_WORKER_RUNS = _assign(_contiguous_runs(_IDS), _NUM_WORKERS)


_D = 768


def _shuffle_body(in_hbm, out_hbm, sem):
    ncores = 2
    wid = lax.axis_index("s") * ncores + lax.axis_index("c")
    for w, runs in enumerate(_WORKER_RUNS):
        if not runs:
            continue

        @pl.when(wid == w)
        def _(runs=runs):
            handles = []
            for j, a, n in runs:
                handles.append(pltpu.make_async_copy(
                    in_hbm.at[:, pl.ds(a * _D, n * _D)],
                    out_hbm.at[:, pl.ds(j * _D, n * _D)],
                    sem,
                ))
            for h in handles:
                h.start()
            for h in handles:
                h.wait()


def kernel(inputs):
    b, t, d = inputs.shape
    flat = inputs.reshape(b, t * d)
    mesh = plsc.VectorSubcoreMesh(core_axis_name="c", subcore_axis_name="s")
    run = functools.partial(
        pl.kernel,
        out_type=jax.ShapeDtypeStruct((b, t * d), inputs.dtype),
        mesh=mesh,
        scratch_types=[pltpu.SemaphoreType.DMA],
    )(_shuffle_body)
    return run(flat).reshape(b, t, d)


# speedup vs baseline: 1.0020x; 1.0020x over previous
import functools

import jax
import jax.numpy as jnp
from jax import lax
from jax.experimental import pallas as pl
from jax.experimental.pallas import tpu as pltpu
from jax.experimental.pallas import tpu_sc as plsc

_IDS = [0, 195, 13, 182, 90, 110, 175, 6, 84, 45, 51, 129, 135, 69, 186, 3,
        9, 42, 48, 87, 93, 126, 132, 152, 192, 25, 81, 155, 159, 41, 53, 157,
        163, 184, 15, 18, 21, 30, 33, 36, 38, 57, 60, 63, 66, 72, 75, 78, 97,
        99, 102, 105, 108, 114, 117, 120, 123, 125, 142, 144, 147, 150, 165,
        167, 180, 188, 190, 1, 2, 4, 5, 7, 8, 10, 11, 12, 14, 16, 17, 19, 20,
        22, 23, 24, 26, 27, 28, 29, 31, 32, 34, 35, 37, 39, 40, 43, 44, 46,
        47, 49, 50, 52, 54, 55, 56, 58, 59, 61, 62, 64, 65, 67, 68, 70, 71,
        73, 74, 76, 77, 79, 80, 82, 83, 85, 86, 88, 89, 91, 92, 94, 95, 96,
        98, 100, 101, 103, 104, 106, 107, 109, 111, 112, 113, 115, 116, 118,
        119, 121, 122, 124, 127, 128, 130, 131, 133, 134, 136, 137, 138, 139,
        140, 141, 143, 145, 146, 148, 149, 151, 153, 154, 156, 158, 160, 161,
        162, 164, 166, 168, 169, 170, 171, 172, 173, 174, 176, 177, 178, 179,
        181, 183, 185, 187, 189, 191, 193, 194]


def _contiguous_runs(ids):
    runs = []
    j = 0
    while j < len(ids):
        a = ids[j]
        n = 1
        while j + n < len(ids) and ids[j + n] == a + n:
            n += 1
        runs.append((j, a, n))
        j += n
    return runs


def _assign(runs, num_workers):
    bins = [[] for _ in range(num_workers)]
    loads = [0] * num_workers
    for run in sorted(runs, key=lambda r: -r[2]):
        w = loads.index(min(loads))
        bins[w].append(run)
        loads[w] += run[2]
    return bins


_NUM_WORKERS = 32
_WORKER_RUNS = _assign(_contiguous_runs(_IDS), _NUM_WORKERS)


_D = 768
_RUNS = _contiguous_runs(_IDS)


def _shuffle_body_tc(in_hbm, out_hbm, sem):
    handles = []
    for j, a, n in _RUNS:
        handles.append(pltpu.make_async_copy(
            in_hbm.at[:, pl.ds(a * _D, n * _D)],
            out_hbm.at[:, pl.ds(j * _D, n * _D)],
            sem,
        ))
    for h in handles:
        h.start()
    for h in handles:
        h.wait()


def kernel(inputs):
    b, t, d = inputs.shape
    flat = inputs.reshape(b, t * d)
    out = pl.pallas_call(
        _shuffle_body_tc,
        out_shape=jax.ShapeDtypeStruct((b, t * d), inputs.dtype),
        in_specs=[pl.BlockSpec(memory_space=pltpu.MemorySpace.HBM)],
        out_specs=pl.BlockSpec(memory_space=pltpu.MemorySpace.HBM),
        scratch_shapes=[pltpu.SemaphoreType.DMA],
    )(flat)
    return out.reshape(b, t, d)


# speedup vs baseline: 6.4307x; 6.4181x over previous
import functools

import jax
import jax.numpy as jnp
from jax import lax
from jax.experimental import pallas as pl
from jax.experimental.pallas import tpu as pltpu
from jax.experimental.pallas import tpu_sc as plsc

_IDS = [0, 195, 13, 182, 90, 110, 175, 6, 84, 45, 51, 129, 135, 69, 186, 3,
        9, 42, 48, 87, 93, 126, 132, 152, 192, 25, 81, 155, 159, 41, 53, 157,
        163, 184, 15, 18, 21, 30, 33, 36, 38, 57, 60, 63, 66, 72, 75, 78, 97,
        99, 102, 105, 108, 114, 117, 120, 123, 125, 142, 144, 147, 150, 165,
        167, 180, 188, 190, 1, 2, 4, 5, 7, 8, 10, 11, 12, 14, 16, 17, 19, 20,
        22, 23, 24, 26, 27, 28, 29, 31, 32, 34, 35, 37, 39, 40, 43, 44, 46,
        47, 49, 50, 52, 54, 55, 56, 58, 59, 61, 62, 64, 65, 67, 68, 70, 71,
        73, 74, 76, 77, 79, 80, 82, 83, 85, 86, 88, 89, 91, 92, 94, 95, 96,
        98, 100, 101, 103, 104, 106, 107, 109, 111, 112, 113, 115, 116, 118,
        119, 121, 122, 124, 127, 128, 130, 131, 133, 134, 136, 137, 138, 139,
        140, 141, 143, 145, 146, 148, 149, 151, 153, 154, 156, 158, 160, 161,
        162, 164, 166, 168, 169, 170, 171, 172, 173, 174, 176, 177, 178, 179,
        181, 183, 185, 187, 189, 191, 193, 194]


def _contiguous_runs(ids):
    runs = []
    j = 0
    while j < len(ids):
        a = ids[j]
        n = 1
        while j + n < len(ids) and ids[j + n] == a + n:
            n += 1
        runs.append((j, a, n))
        j += n
    return runs


def _assign(runs, num_workers):
    bins = [[] for _ in range(num_workers)]
    loads = [0] * num_workers
    for run in sorted(runs, key=lambda r: -r[2]):
        w = loads.index(min(loads))
        bins[w].append(run)
        loads[w] += run[2]
    return bins


_NUM_WORKERS = 32
_WORKER_RUNS = _assign(_contiguous_runs(_IDS), _NUM_WORKERS)


_D = 768
_RUNS = _contiguous_runs(_IDS)


def _gather_body(ids_ref, in_ref, out_ref):
    out_ref[...] = in_ref[...]


def kernel(inputs):
    b, t, d = inputs.shape
    flat = inputs.reshape(b, t * d)
    ids = jnp.asarray(_IDS, dtype=jnp.int32)
    grid_spec = pltpu.PrefetchScalarGridSpec(
        num_scalar_prefetch=1,
        grid=(t,),
        in_specs=[pl.BlockSpec((b, d), lambda j, ids_ref: (0, ids_ref[j]))],
        out_specs=pl.BlockSpec((b, d), lambda j, ids_ref: (0, j)),
    )
    out = pl.pallas_call(
        _gather_body,
        grid_spec=grid_spec,
        out_shape=jax.ShapeDtypeStruct((b, t * d), inputs.dtype),
        compiler_params=pltpu.CompilerParams(
            dimension_semantics=("arbitrary",),
        ),
    )(ids, flat)
    return out.reshape(b, t, d)


# speedup vs baseline: 14.5871x; 2.2684x over previous
import functools

import jax
import jax.numpy as jnp
from jax import lax
from jax.experimental import pallas as pl
from jax.experimental.pallas import tpu as pltpu
from jax.experimental.pallas import tpu_sc as plsc

_IDS = [0, 195, 13, 182, 90, 110, 175, 6, 84, 45, 51, 129, 135, 69, 186, 3,
        9, 42, 48, 87, 93, 126, 132, 152, 192, 25, 81, 155, 159, 41, 53, 157,
        163, 184, 15, 18, 21, 30, 33, 36, 38, 57, 60, 63, 66, 72, 75, 78, 97,
        99, 102, 105, 108, 114, 117, 120, 123, 125, 142, 144, 147, 150, 165,
        167, 180, 188, 190, 1, 2, 4, 5, 7, 8, 10, 11, 12, 14, 16, 17, 19, 20,
        22, 23, 24, 26, 27, 28, 29, 31, 32, 34, 35, 37, 39, 40, 43, 44, 46,
        47, 49, 50, 52, 54, 55, 56, 58, 59, 61, 62, 64, 65, 67, 68, 70, 71,
        73, 74, 76, 77, 79, 80, 82, 83, 85, 86, 88, 89, 91, 92, 94, 95, 96,
        98, 100, 101, 103, 104, 106, 107, 109, 111, 112, 113, 115, 116, 118,
        119, 121, 122, 124, 127, 128, 130, 131, 133, 134, 136, 137, 138, 139,
        140, 141, 143, 145, 146, 148, 149, 151, 153, 154, 156, 158, 160, 161,
        162, 164, 166, 168, 169, 170, 171, 172, 173, 174, 176, 177, 178, 179,
        181, 183, 185, 187, 189, 191, 193, 194]


def _contiguous_runs(ids):
    runs = []
    j = 0
    while j < len(ids):
        a = ids[j]
        n = 1
        while j + n < len(ids) and ids[j + n] == a + n:
            n += 1
        runs.append((j, a, n))
        j += n
    return runs


def _assign(runs, num_workers):
    bins = [[] for _ in range(num_workers)]
    loads = [0] * num_workers
    for run in sorted(runs, key=lambda r: -r[2]):
        w = loads.index(min(loads))
        bins[w].append(run)
        loads[w] += run[2]
    return bins


_NUM_WORKERS = 32
_WORKER_RUNS = _assign(_contiguous_runs(_IDS), _NUM_WORKERS)


_RUNS = _contiguous_runs(_IDS)
_BB = 4


def _permute_body(in_ref, out_ref):
    for j, a, n in _RUNS:
        out_ref[:, j:j + n, :] = in_ref[:, a:a + n, :]


def kernel(inputs):
    b, t, d = inputs.shape
    out = pl.pallas_call(
        _permute_body,
        grid=(b // _BB,),
        in_specs=[pl.BlockSpec((_BB, t, d), lambda i: (i, 0, 0))],
        out_specs=pl.BlockSpec((_BB, t, d), lambda i: (i, 0, 0)),
        out_shape=jax.ShapeDtypeStruct((b, t, d), inputs.dtype),
        compiler_params=pltpu.CompilerParams(
            dimension_semantics=("arbitrary",),
        ),
    )(inputs)
    return out
